# baseline (speedup 1.0000x reference)
import jax
import jax.numpy as jnp
from jax import lax
from jax.experimental import pallas as pl
from jax.experimental.pallas import tpu as pltpu
from jax.experimental.pallas import tpu_sc as plsc

_EPS = jnp.finfo(jnp.float32).eps

_R = 262144
_NF = 32
_NP = 64
_G = 16

_NC, _NS, _L = 2, 16, 16
_NW = _NC * _NS
_ROWS_PER_W = _R // _NW
_GR = 256
_NGRP = _ROWS_PER_W // _GR
_STRIPS = _GR // _G

_CT = _NF + 1
_CH = _NP + 1


def _sc_body(t_hbm, w_hbm, th_hbm, wh_hbm, out_hbm,
             wb0, ob0, th0b0, tmb0,
             wb1, ob1, th0b1, tmb1,
             tb, thb, whb, sin0, sin1):
    bufs = ((wb0, ob0, th0b0, tmb0, sin0),
            (wb1, ob1, th0b1, tmb1, sin1))
    rows = lax.iota(jnp.int32, _G)
    wid = lax.axis_index("s") * _NC + lax.axis_index("c")
    base = wid * _ROWS_PER_W

    def in_copies(g, b):
        wb, _, th0b, tmb, sem = bufs[b]
        r0 = base + g * _GR
        return (
            (w_hbm.at[:, pl.ds(r0, _GR)], wb, sem),
            (th_hbm.at[pl.ds(0, 1), pl.ds(r0, _GR)], th0b, sem),
            (t_hbm.at[pl.ds(_NF, 1), pl.ds(r0, _GR)], tmb, sem),
        )

    def start_in(g, b):
        for src, dst, sem in in_copies(g, b):
            pltpu.async_copy(src, dst, sem)

    def wait_in(g, b):
        for src, dst, sem in in_copies(g, b):
            pltpu.make_async_copy(src, dst, sem).wait()

    def fast_strip(wb, ob, s):
        c0 = s * _G
        for c in range(_NF):
            wv = wb[c, pl.ds(c0, _G)]
            d = jnp.maximum(wv, 0.0)
            ob[c, pl.ds(c0, _G)] = d * d / (wv + _EPS)

    def compute_group(g, b):
        wb, ob, th0b, tmb, _ = bufs[b]

        def fast_group():
            def sbody(s, _):
                fast_strip(wb, ob, s)
                return 0
            lax.fori_loop(0, _STRIPS, sbody, 0)

        def slow_group():
            r0 = base + g * _GR
            pltpu.sync_copy(t_hbm.at[:, pl.ds(r0, _GR)], tb)
            pltpu.sync_copy(th_hbm.at[:, pl.ds(r0, _GR)], thb)
            pltpu.sync_copy(wh_hbm.at[:, pl.ds(r0, _GR)], whb)

            def strip_body(s, _):
                c0 = s * _G
                scol = rows + c0

                def masks(p_l, p_r, tj):
                    thl = plsc.load_gather(
                        thb, [jnp.minimum(p_l, _NP), scol])
                    thr = plsc.load_gather(
                        thb, [jnp.minimum(p_r, _NP), scol])
                    m_l = (thl < tj) & (p_l < _CH)
                    m_r = (thr <= tj) & (p_r < _CH)
                    return m_l, m_r

                def advance(carry, tj):
                    m_l, m_r = masks(carry[0], carry[1], tj)

                    def cond(c):
                        return jnp.any(c[4] | c[5])

                    def body(c):
                        p_l, p_r, s_l, s_r, m_l, m_r = c
                        wl = plsc.load_gather(
                            whb, [jnp.minimum(p_l, _NP - 1), scol])
                        wr = plsc.load_gather(
                            whb, [jnp.minimum(p_r, _NP - 1), scol])
                        s_l = s_l + jnp.where(m_l & (p_l < _NP), wl, 0.0)
                        s_r = s_r + jnp.where(m_r & (p_r < _NP), wr, 0.0)
                        p_l = p_l + m_l.astype(jnp.int32)
                        p_r = p_r + m_r.astype(jnp.int32)
                        m_l, m_r = masks(p_l, p_r, tj)
                        return (p_l, p_r, s_l, s_r, m_l, m_r)

                    c = lax.while_loop(
                        cond, body, (carry[0], carry[1], carry[2],
                                     carry[3], m_l, m_r))
                    return c[0], c[1], c[2], c[3]

                def j_body(j, carry):
                    p_l, p_r, s_l, s_r, prev_sl = carry
                    tj = tb[j, pl.ds(c0, _G)]
                    p_l, p_r, s_l, s_r = advance((p_l, p_r, s_l, s_r), tj)
                    jm1 = j - 1
                    wv = wb[jm1, pl.ds(c0, _G)]
                    d = jnp.maximum(wv - (s_r - prev_sl), 0.0)
                    ob[jm1, pl.ds(c0, _G)] = d * d / (wv + _EPS)
                    return (p_l, p_r, s_l, s_r, s_l)

                def strip_slow():
                    zf = jnp.zeros((_G,), jnp.float32)
                    zi = jnp.zeros((_G,), jnp.int32)
                    t0 = tb[0, pl.ds(c0, _G)]
                    p_l, p_r, s_l, s_r = advance((zi, zi, zf, zf), t0)
                    lax.fori_loop(1, _CT, j_body, (p_l, p_r, s_l, s_r, s_l))

                def strip_fast():
                    fast_strip(wb, ob, s)

                th0 = th0b[0, pl.ds(c0, _G)]
                tmax = tmb[0, pl.ds(c0, _G)]
                lax.cond(jnp.all(th0 > tmax), strip_fast, strip_slow)
                return 0

            lax.fori_loop(0, _STRIPS, strip_body, 0)

        dis = jnp.ones((_G,), jnp.bool_)
        for s in range(_STRIPS):
            dis = dis & (th0b[0, pl.ds(s * _G, _G)]
                         > tmb[0, pl.ds(s * _G, _G)])
        lax.cond(jnp.all(dis), fast_group, slow_group)

    start_in(0, 0)
    start_in(1, 1)

    def outer(i, _):
        g0 = 2 * i
        for b in range(2):
            g = g0 + b
            wait_in(g, b)
            compute_group(g, b)
            ob = bufs[b][1]
            r0 = base + g * _GR
            pltpu.sync_copy(ob, out_hbm.at[:, pl.ds(r0, _GR)])

            @pl.when(g + 2 < _NGRP)
            def _():
                start_in(g + 2, b)
        return 0

    lax.fori_loop(0, _NGRP // 2, outer, 0)


@jax.jit
def kernel(t, w, t_hat, w_hat):
    mesh = plsc.VectorSubcoreMesh(core_axis_name="c", subcore_axis_name="s",
                                  num_cores=_NC, num_subcores=_NS)
    f32 = jnp.float32
    run = pl.kernel(
        _sc_body,
        out_type=jax.ShapeDtypeStruct((_NF, _R), f32),
        mesh=mesh,
        scratch_types=[
            pltpu.VMEM((_NF, _GR), f32),
            pltpu.VMEM((_NF, _GR), f32),
            pltpu.VMEM((1, _GR), f32),
            pltpu.VMEM((1, _GR), f32),
            pltpu.VMEM((_NF, _GR), f32),
            pltpu.VMEM((_NF, _GR), f32),
            pltpu.VMEM((1, _GR), f32),
            pltpu.VMEM((1, _GR), f32),
            pltpu.VMEM((_CT, _GR), f32),
            pltpu.VMEM((_CH, _GR), f32),
            pltpu.VMEM((_NP, _GR), f32),
            pltpu.SemaphoreType.DMA,
            pltpu.SemaphoreType.DMA,
        ],
        compiler_params=pltpu.CompilerParams(needs_layout_passes=False),
    )
    out_t = run(t.T, w.T, t_hat.T, w_hat.T)
    return out_t.T


# speedup vs baseline: 2.9669x; 2.9669x over previous
import jax
import jax.numpy as jnp
from jax import lax
from jax.experimental import pallas as pl
from jax.experimental.pallas import tpu as pltpu
from jax.experimental.pallas import tpu_sc as plsc

_EPS = jnp.finfo(jnp.float32).eps

_R = 262144
_NF = 32
_NP = 64
_G = 16

_NC, _NS, _L = 2, 16, 16
_NW = _NC * _NS
_ROWS_PER_W = _R // _NW
_GR = 128
_NGRP = _ROWS_PER_W // _GR
_STRIPS = _GR // _G

_CT = _NF + 1
_CH = _NP + 1


def _sc_body(t_hbm, w_hbm, th_hbm, wh_hbm, out_hbm,
             wb0, ob0, th0b0, tmb0,
             wb1, ob1, th0b1, tmb1,
             tb, thb, whb, sin0, sin1):
    bufs = ((wb0, ob0, th0b0, tmb0, sin0),
            (wb1, ob1, th0b1, tmb1, sin1))
    rows = lax.iota(jnp.int32, _G)
    wid = lax.axis_index("s") * _NC + lax.axis_index("c")
    base = wid * _ROWS_PER_W

    def in_copies(g, b):
        wb, _, th0b, tmb, sem = bufs[b]
        r0 = base + g * _GR
        return (
            (w_hbm.at[:, pl.ds(r0, _GR)], wb, sem),
            (th_hbm.at[pl.ds(0, 1), pl.ds(r0, _GR)], th0b, sem),
            (t_hbm.at[pl.ds(_NF, 1), pl.ds(r0, _GR)], tmb, sem),
        )

    def start_in(g, b):
        for src, dst, sem in in_copies(g, b):
            pltpu.async_copy(src, dst, sem)

    def wait_in(g, b):
        for src, dst, sem in in_copies(g, b):
            pltpu.make_async_copy(src, dst, sem).wait()

    def fast_strip(wb, ob, s):
        c0 = s * _G
        for c in range(_NF):
            wv = wb[c, pl.ds(c0, _G)]
            d = jnp.maximum(wv, 0.0)
            ob[c, pl.ds(c0, _G)] = d * d / (wv + _EPS)

    def compute_group(g, b):
        wb, ob, th0b, tmb, _ = bufs[b]

        def fast_group():
            def sbody(s, _):
                fast_strip(wb, ob, s)
                return 0
            lax.fori_loop(0, _STRIPS, sbody, 0)

        def slow_group():
            r0 = base + g * _GR
            pltpu.sync_copy(t_hbm.at[:, pl.ds(r0, _GR)], tb)
            pltpu.sync_copy(th_hbm.at[:, pl.ds(r0, _GR)], thb)
            pltpu.sync_copy(wh_hbm.at[:, pl.ds(r0, _GR)], whb)

            def strip_body(s, _):
                c0 = s * _G
                scol = rows + c0

                def masks(p_l, p_r, tj):
                    thl = plsc.load_gather(
                        thb, [jnp.minimum(p_l, _NP), scol])
                    thr = plsc.load_gather(
                        thb, [jnp.minimum(p_r, _NP), scol])
                    m_l = (thl < tj) & (p_l < _CH)
                    m_r = (thr <= tj) & (p_r < _CH)
                    return m_l, m_r

                def advance(carry, tj):
                    m_l, m_r = masks(carry[0], carry[1], tj)

                    def cond(c):
                        return jnp.any(c[4] | c[5])

                    def body(c):
                        p_l, p_r, s_l, s_r, m_l, m_r = c
                        wl = plsc.load_gather(
                            whb, [jnp.minimum(p_l, _NP - 1), scol])
                        wr = plsc.load_gather(
                            whb, [jnp.minimum(p_r, _NP - 1), scol])
                        s_l = s_l + jnp.where(m_l & (p_l < _NP), wl, 0.0)
                        s_r = s_r + jnp.where(m_r & (p_r < _NP), wr, 0.0)
                        p_l = p_l + m_l.astype(jnp.int32)
                        p_r = p_r + m_r.astype(jnp.int32)
                        m_l, m_r = masks(p_l, p_r, tj)
                        return (p_l, p_r, s_l, s_r, m_l, m_r)

                    c = lax.while_loop(
                        cond, body, (carry[0], carry[1], carry[2],
                                     carry[3], m_l, m_r))
                    return c[0], c[1], c[2], c[3]

                def j_body(j, carry):
                    p_l, p_r, s_l, s_r, prev_sl = carry
                    tj = tb[j, pl.ds(c0, _G)]
                    p_l, p_r, s_l, s_r = advance((p_l, p_r, s_l, s_r), tj)
                    jm1 = j - 1
                    wv = wb[jm1, pl.ds(c0, _G)]
                    d = jnp.maximum(wv - (s_r - prev_sl), 0.0)
                    ob[jm1, pl.ds(c0, _G)] = d * d / (wv + _EPS)
                    return (p_l, p_r, s_l, s_r, s_l)

                def strip_slow():
                    zf = jnp.zeros((_G,), jnp.float32)
                    zi = jnp.zeros((_G,), jnp.int32)
                    t0 = tb[0, pl.ds(c0, _G)]
                    p_l, p_r, s_l, s_r = advance((zi, zi, zf, zf), t0)
                    lax.fori_loop(1, _CT, j_body, (p_l, p_r, s_l, s_r, s_l))

                def strip_fast():
                    fast_strip(wb, ob, s)

                th0 = th0b[0, pl.ds(c0, _G)]
                tmax = tmb[0, pl.ds(c0, _G)]
                lax.cond(jnp.all(th0 > tmax), strip_fast, strip_slow)
                return 0

            lax.fori_loop(0, _STRIPS, strip_body, 0)

        dis = jnp.ones((_G,), jnp.bool_)
        for s in range(_STRIPS):
            dis = dis & (th0b[0, pl.ds(s * _G, _G)]
                         > tmb[0, pl.ds(s * _G, _G)])
        lax.cond(jnp.all(dis), fast_group, slow_group)

    start_in(0, 0)
    start_in(1, 1)

    def outer(i, _):
        g0 = 2 * i
        for b in range(2):
            g = g0 + b
            wait_in(g, b)
            compute_group(g, b)
            ob = bufs[b][1]
            r0 = base + g * _GR
            pltpu.sync_copy(ob, out_hbm.at[:, pl.ds(r0, _GR)])

            @pl.when(g + 2 < _NGRP)
            def _():
                start_in(g + 2, b)
        return 0

    lax.fori_loop(0, _NGRP // 2, outer, 0)


@jax.jit
def kernel(t, w, t_hat, w_hat):
    mesh = plsc.VectorSubcoreMesh(core_axis_name="c", subcore_axis_name="s",
                                  num_cores=_NC, num_subcores=_NS)
    f32 = jnp.float32
    run = pl.kernel(
        _sc_body,
        out_type=jax.ShapeDtypeStruct((_NF, _R), f32),
        mesh=mesh,
        scratch_types=[
            pltpu.VMEM((_NF, _GR), f32),
            pltpu.VMEM((_NF, _GR), f32),
            pltpu.VMEM((1, _GR), f32),
            pltpu.VMEM((1, _GR), f32),
            pltpu.VMEM((_NF, _GR), f32),
            pltpu.VMEM((_NF, _GR), f32),
            pltpu.VMEM((1, _GR), f32),
            pltpu.VMEM((1, _GR), f32),
            pltpu.VMEM((_CT, _GR), f32),
            pltpu.VMEM((_CH, _GR), f32),
            pltpu.VMEM((_NP, _GR), f32),
            pltpu.SemaphoreType.DMA,
            pltpu.SemaphoreType.DMA,
        ],
        compiler_params=pltpu.CompilerParams(needs_layout_passes=False),
    )
    out_t = run(t.T, w.T, t_hat.T, w_hat.T)
    return out_t.T
